# Initial kernel scaffold; baseline (speedup 1.0000x reference)
import jax
import jax.numpy as jnp
from jax.experimental import pallas as pl


def kernel(toLearn):
    raise NotImplementedError("write your pallas kernel here")


# speedup vs baseline: 1.0262x; 1.0262x over previous
import jax
import jax.numpy as jnp
from jax.experimental import pallas as pl

_N = 2048
_K = 150
_NCOMP = 784
_BLK = 256


def _knn_select_body(d2_ref, a_ref):
    d2 = d2_ref[...]
    bits = jax.lax.bitcast_convert_type(d2, jnp.int32)
    skey = jnp.where(bits < 0, bits ^ jnp.int32(0x7FFFFFFF), bits)

    k0 = jnp.full((_BLK, 1), _K, jnp.int32)
    cnt_neg = jnp.sum((skey < 0).astype(jnp.int32), axis=1, keepdims=True)
    use_neg = k0 <= cnt_neg
    prefix = jnp.where(use_neg, jnp.int32(-(2**31)), jnp.int32(0))
    k_rem = jnp.where(use_neg, k0, k0 - cnt_neg)

    for b in range(30, -1, -1):
        match0 = (skey >> b) == (prefix >> b)
        cnt0 = jnp.sum(match0.astype(jnp.int32), axis=1, keepdims=True)
        take1 = k_rem > cnt0
        prefix = jnp.where(take1, prefix | jnp.int32(1 << b), prefix)
        k_rem = jnp.where(take1, k_rem - cnt0, k_rem)

    t = prefix
    lt = skey < t
    tie = skey == t
    need = k0 - jnp.sum(lt.astype(jnp.int32), axis=1, keepdims=True)
    col = jax.lax.broadcasted_iota(jnp.int32, (_BLK, _N), 1)
    ipref = jnp.zeros((_BLK, 1), jnp.int32)
    for b in range(10, -1, -1):
        m0 = tie & ((col >> b) == (ipref >> b))
        cnt0 = jnp.sum(m0.astype(jnp.int32), axis=1, keepdims=True)
        take1 = need > cnt0
        ipref = jnp.where(take1, ipref | jnp.int32(1 << b), ipref)
        need = jnp.where(take1, need - cnt0, need)

    sel = lt | (tie & (col <= ipref))
    a_ref[...] = sel.astype(jnp.float32)


def _knn_adjacency(d2, interpret=False):
    n = d2.shape[0]
    return pl.pallas_call(
        _knn_select_body,
        grid=(n // _BLK,),
        in_specs=[pl.BlockSpec((_BLK, n), lambda i: (i, 0))],
        out_specs=pl.BlockSpec((_BLK, n), lambda i: (i, 0)),
        out_shape=jax.ShapeDtypeStruct((n, n), jnp.float32),
        interpret=interpret,
    )(d2)


def kernel(toLearn):
    flat = toLearn.reshape(toLearn.shape[0], -1)
    n = flat.shape[0]
    sq = jnp.sum(flat * flat, axis=1)
    d2 = sq[:, None] + sq[None, :] - 2.0 * (flat @ flat.T)
    d2 = d2 + jnp.eye(n, dtype=flat.dtype) * 1e12

    A = _knn_adjacency(d2)

    W = 0.5 * (A + A.T)
    deg = jnp.sum(W, axis=1)
    dd = jnp.sqrt(deg)
    L = jnp.eye(n, dtype=jnp.float32) - (W / dd[:, None]) / dd[None, :]
    evals, evecs = jnp.linalg.eigh(L)
    emb = evecs[:, 1:_NCOMP + 1] / dd[:, None]
    max_abs_row = jnp.argmax(jnp.abs(emb), axis=0)
    signs = jnp.sign(emb[max_abs_row, jnp.arange(emb.shape[1])])
    signs = jnp.where(signs == 0, 1.0, signs)
    emb = jax.lax.stop_gradient(emb * signs[None, :])
    return emb.reshape(n, 1, 28, 28).astype(jnp.float32)
